# baseline (speedup 1.0000x reference)
import functools

import numpy as np
import jax
import jax.numpy as jnp
from jax import lax
from jax.experimental import pallas as pl
from jax.experimental.pallas import tpu as pltpu
from jax.experimental.pallas import tpu_sc as plsc

BATCH = 32
SEQ = 2048
D = 128
SCALE = float(np.sqrt(128.0))

_info = plsc.get_sparse_core_info()
NC, NS, L = _info.num_cores, _info.num_subcores, _info.num_lanes
NW = NC * NS
BAND = SEQ // NW


def _pos_encoding() -> np.ndarray:
    positions = np.arange(SEQ)[:, np.newaxis].astype(np.float64)
    depths = np.arange(D)[np.newaxis, :] / D
    angle_rates = 1.0 / (10000.0 ** depths)
    pe = positions * angle_rates
    pe[:, 1::2] = np.cos(pe[:, 1::2])
    pe[:, 0::2] = np.sin(pe[:, 0::2])
    return pe.astype(np.float32)


_POS = _pos_encoding()

_mesh = plsc.VectorSubcoreMesh(core_axis_name="c", subcore_axis_name="s")


@functools.partial(
    pl.kernel,
    mesh=_mesh,
    out_type=jax.ShapeDtypeStruct((BATCH, SEQ, D), jnp.float32),
    scratch_types=[
        pltpu.VMEM((BATCH, BAND), jnp.int32),
        pltpu.VMEM((BAND, D), jnp.float32),
        pltpu.VMEM((BAND, D), jnp.float32),
        pltpu.SemaphoreType.DMA,
    ],
)
def _emb_kernel(xp_hbm, pos_hbm, table_hbm, out_hbm, idx_v, pos_v, rows_v, sem):
    wid = lax.axis_index("s") * NC + lax.axis_index("c")
    s0 = wid * BAND
    pltpu.sync_copy(xp_hbm.at[wid], idx_v)
    pltpu.sync_copy(pos_hbm.at[pl.ds(s0, BAND), :], pos_v)

    def body_b(b, carry):
        pltpu.async_copy(table_hbm.at[idx_v.at[b]], rows_v, sem).wait()

        def body_i(i, c):
            for j in range(D // L):
                sl = pl.ds(j * L, L)
                rows_v[i, sl] = rows_v[i, sl] * SCALE + pos_v[i, sl]
            return c

        lax.fori_loop(0, BAND, body_i, 0)
        pltpu.sync_copy(rows_v, out_hbm.at[b, pl.ds(s0, BAND), :])
        return carry

    lax.fori_loop(0, BATCH, body_b, 0)


def kernel(x, table):
    pos = jnp.asarray(_POS)
    xp = x.reshape(BATCH, NW, BAND).transpose(1, 0, 2)
    return _emb_kernel(xp, pos, table)


# speedup vs baseline: 1.4858x; 1.4858x over previous
import functools

import numpy as np
import jax
import jax.numpy as jnp
from jax import lax
from jax.experimental import pallas as pl
from jax.experimental.pallas import tpu as pltpu
from jax.experimental.pallas import tpu_sc as plsc

BATCH = 32
SEQ = 2048
D = 128
SCALE = float(np.sqrt(128.0))

_info = plsc.get_sparse_core_info()
NC, NS, L = _info.num_cores, _info.num_subcores, _info.num_lanes
NW = NC * NS
BAND = SEQ // NW


def _pos_encoding() -> np.ndarray:
    positions = np.arange(SEQ)[:, np.newaxis].astype(np.float64)
    depths = np.arange(D)[np.newaxis, :] / D
    angle_rates = 1.0 / (10000.0 ** depths)
    pe = positions * angle_rates
    pe[:, 1::2] = np.cos(pe[:, 1::2])
    pe[:, 0::2] = np.sin(pe[:, 0::2])
    return pe.astype(np.float32)


_POS = _pos_encoding()

_mesh = plsc.VectorSubcoreMesh(core_axis_name="c", subcore_axis_name="s")


@functools.partial(
    pl.kernel,
    mesh=_mesh,
    out_type=jax.ShapeDtypeStruct((BATCH, SEQ, D), jnp.float32),
    scratch_types=[
        pltpu.VMEM((BATCH, BAND), jnp.int32),
        pltpu.VMEM((BAND, D), jnp.float32),
        pltpu.VMEM((BAND, D), jnp.float32),
        pltpu.VMEM((BAND, D), jnp.float32),
        pltpu.SemaphoreType.DMA,
        pltpu.SemaphoreType.DMA,
        pltpu.SemaphoreType.DMA,
        pltpu.SemaphoreType.DMA,
        pltpu.SemaphoreType.DMA,
        pltpu.SemaphoreType.DMA,
    ],
)
def _emb_kernel(xp_hbm, pos_hbm, table_hbm, out_hbm,
                idx_v, pos_v, rows0, rows1,
                sem_i, sem_p, sem_g0, sem_g1, sem_o0, sem_o1):
    wid = lax.axis_index("s") * NC + lax.axis_index("c")
    s0 = wid * BAND
    rows = (rows0, rows1)
    sem_g = (sem_g0, sem_g1)
    sem_o = (sem_o0, sem_o1)

    cp_i = pltpu.async_copy(xp_hbm.at[wid], idx_v, sem_i)
    cp_p = pltpu.async_copy(pos_hbm.at[pl.ds(s0, BAND), :], pos_v, sem_p)
    cp_i.wait()
    gath = [None, None]
    outc = [None, None]
    gath[0] = pltpu.async_copy(table_hbm.at[idx_v.at[0]], rows[0], sem_g[0])
    cp_p.wait()

    for b in range(BATCH):
        s = b & 1
        if b + 1 < BATCH:
            if outc[1 - s] is not None:
                outc[1 - s].wait()
            gath[1 - s] = pltpu.async_copy(
                table_hbm.at[idx_v.at[b + 1]], rows[1 - s], sem_g[1 - s])
        gath[s].wait()
        buf = rows[s]

        def body_i(i, c, buf=buf):
            for j in range(D // L):
                sl = pl.ds(j * L, L)
                buf[i, sl] = buf[i, sl] * SCALE + pos_v[i, sl]
            return c

        lax.fori_loop(0, BAND, body_i, 0)
        outc[s] = pltpu.async_copy(
            buf, out_hbm.at[b, pl.ds(s0, BAND), :], sem_o[s])
    outc[0].wait()
    outc[1].wait()


def kernel(x, table):
    pos = jnp.asarray(_POS)
    xp = x.reshape(BATCH, NW, BAND).transpose(1, 0, 2)
    return _emb_kernel(xp, pos, table)


# speedup vs baseline: 1.8115x; 1.2192x over previous
import functools

import numpy as np
import jax
import jax.numpy as jnp
from jax import lax
from jax.experimental import pallas as pl
from jax.experimental.pallas import tpu as pltpu
from jax.experimental.pallas import tpu_sc as plsc

BATCH = 32
SEQ = 2048
D = 128
SCALE = float(np.sqrt(128.0))

_info = plsc.get_sparse_core_info()
NC, NS, L = _info.num_cores, _info.num_subcores, _info.num_lanes
NW = NC * NS
BAND = SEQ // NW
CH_B = 4
N_CHUNK = BATCH // CH_B
CH_ROWS = CH_B * BAND


def _pos_encoding() -> np.ndarray:
    positions = np.arange(SEQ)[:, np.newaxis].astype(np.float64)
    depths = np.arange(D)[np.newaxis, :] / D
    angle_rates = 1.0 / (10000.0 ** depths)
    pe = positions * angle_rates
    pe[:, 1::2] = np.cos(pe[:, 1::2])
    pe[:, 0::2] = np.sin(pe[:, 0::2])
    return pe.astype(np.float32)


_POS = _pos_encoding()

_mesh = plsc.VectorSubcoreMesh(core_axis_name="c", subcore_axis_name="s")


@functools.partial(
    pl.kernel,
    mesh=_mesh,
    out_type=jax.ShapeDtypeStruct((BATCH, SEQ, D), jnp.float32),
    scratch_types=[
        pltpu.VMEM((BATCH * BAND,), jnp.int32),
        pltpu.VMEM((BAND, D), jnp.float32),
        pltpu.VMEM((CH_ROWS, D), jnp.float32),
        pltpu.VMEM((CH_ROWS, D), jnp.float32),
        pltpu.SemaphoreType.DMA,
        pltpu.SemaphoreType.DMA,
        pltpu.SemaphoreType.DMA,
        pltpu.SemaphoreType.DMA,
        pltpu.SemaphoreType.DMA,
        pltpu.SemaphoreType.DMA,
    ],
)
def _emb_kernel(xp_hbm, pos_hbm, table_hbm, out_hbm,
                idx_v, pos_v, rows0, rows1,
                sem_i, sem_p, sem_g0, sem_g1, sem_o0, sem_o1):
    wid = lax.axis_index("s") * NC + lax.axis_index("c")
    s0 = wid * BAND
    rows = (rows0, rows1)
    sem_g = (sem_g0, sem_g1)
    sem_o = (sem_o0, sem_o1)

    cp_i = pltpu.async_copy(xp_hbm.at[wid], idx_v, sem_i)
    cp_p = pltpu.async_copy(pos_hbm.at[pl.ds(s0, BAND), :], pos_v, sem_p)
    cp_i.wait()
    gath = [None, None]
    outc = [[], []]
    gath[0] = pltpu.async_copy(
        table_hbm.at[idx_v.at[pl.ds(0, CH_ROWS)]], rows[0], sem_g[0])
    cp_p.wait()

    for c in range(N_CHUNK):
        s = c & 1
        if c + 1 < N_CHUNK:
            for h in outc[1 - s]:
                h.wait()
            outc[1 - s] = []
            gath[1 - s] = pltpu.async_copy(
                table_hbm.at[idx_v.at[pl.ds((c + 1) * CH_ROWS, CH_ROWS)]],
                rows[1 - s], sem_g[1 - s])
        gath[s].wait()
        buf = rows[s]

        def body_i(i, carry, buf=buf):
            pv = [pos_v[i, pl.ds(j * L, L)] for j in range(D // L)]
            for bb in range(CH_B):
                r = bb * BAND + i
                for j in range(D // L):
                    sl = pl.ds(j * L, L)
                    buf[r, sl] = buf[r, sl] * SCALE + pv[j]
            return carry

        lax.fori_loop(0, BAND, body_i, 0)

        for bb in range(CH_B):
            outc[s].append(pltpu.async_copy(
                buf.at[pl.ds(bb * BAND, BAND)],
                out_hbm.at[c * CH_B + bb, pl.ds(s0, BAND), :],
                sem_o[s]))
    for hs in outc:
        for h in hs:
            h.wait()


def kernel(x, table):
    pos = jnp.asarray(_POS)
    xp = x.reshape(BATCH, NW, BAND).transpose(1, 0, 2).reshape(NW, BATCH * BAND)
    return _emb_kernel(xp, pos, table)


# speedup vs baseline: 1.8476x; 1.0199x over previous
import functools

import numpy as np
import jax
import jax.numpy as jnp
from jax import lax
from jax.experimental import pallas as pl
from jax.experimental.pallas import tpu as pltpu
from jax.experimental.pallas import tpu_sc as plsc

BATCH = 32
SEQ = 2048
D = 128
SCALE = float(np.sqrt(128.0))

_info = plsc.get_sparse_core_info()
NC, NS, L = _info.num_cores, _info.num_subcores, _info.num_lanes
NW = NC * NS
BAND = SEQ // NW
CH_B = 4
N_CHUNK = BATCH // CH_B
CH_ROWS = CH_B * BAND


def _pos_encoding() -> np.ndarray:
    positions = np.arange(SEQ)[:, np.newaxis].astype(np.float64)
    depths = np.arange(D)[np.newaxis, :] / D
    angle_rates = 1.0 / (10000.0 ** depths)
    pe = positions * angle_rates
    pe[:, 1::2] = np.cos(pe[:, 1::2])
    pe[:, 0::2] = np.sin(pe[:, 0::2])
    return pe.astype(np.float32)


_POS = _pos_encoding()

_mesh = plsc.VectorSubcoreMesh(core_axis_name="c", subcore_axis_name="s")


@functools.partial(
    pl.kernel,
    mesh=_mesh,
    out_type=jax.ShapeDtypeStruct((BATCH, SEQ, D), jnp.float32),
    scratch_types=[
        pltpu.VMEM((BATCH * BAND,), jnp.int32),
        pltpu.VMEM((BAND, D), jnp.float32),
        pltpu.VMEM((CH_ROWS, D), jnp.float32),
        pltpu.VMEM((CH_ROWS, D), jnp.float32),
        pltpu.VMEM((CH_ROWS, D), jnp.float32),
        pltpu.SemaphoreType.DMA,
        pltpu.SemaphoreType.DMA,
        pltpu.SemaphoreType.DMA,
        pltpu.SemaphoreType.DMA,
        pltpu.SemaphoreType.DMA,
        pltpu.SemaphoreType.DMA,
        pltpu.SemaphoreType.DMA,
        pltpu.SemaphoreType.DMA,
    ],
)
def _emb_kernel(xp_hbm, pos_hbm, table_hbm, out_hbm,
                idx_v, pos_v, rows0, rows1, rows2,
                sem_i, sem_p, sem_g0, sem_g1, sem_g2, sem_o0, sem_o1, sem_o2):
    wid = lax.axis_index("s") * NC + lax.axis_index("c")
    s0 = wid * BAND
    rows = (rows0, rows1, rows2)
    sem_g = (sem_g0, sem_g1, sem_g2)
    sem_o = (sem_o0, sem_o1, sem_o2)
    NBUF = 3

    cp_i = pltpu.async_copy(xp_hbm.at[wid], idx_v, sem_i)
    cp_p = pltpu.async_copy(pos_hbm.at[pl.ds(s0, BAND), :], pos_v, sem_p)
    cp_i.wait()
    gath = [None, None, None]
    outc = [[], [], []]
    gath[0] = pltpu.async_copy(
        table_hbm.at[idx_v.at[pl.ds(0, CH_ROWS)]], rows[0], sem_g[0])
    gath[1] = pltpu.async_copy(
        table_hbm.at[idx_v.at[pl.ds(CH_ROWS, CH_ROWS)]], rows[1], sem_g[1])
    cp_p.wait()

    for c in range(N_CHUNK):
        s = c % NBUF
        if c + 2 < N_CHUNK:
            s2 = (c + 2) % NBUF
            for h in outc[s2]:
                h.wait()
            outc[s2] = []
            gath[s2] = pltpu.async_copy(
                table_hbm.at[idx_v.at[pl.ds((c + 2) * CH_ROWS, CH_ROWS)]],
                rows[s2], sem_g[s2])
        gath[s].wait()
        buf = rows[s]

        def body_i(i, carry, buf=buf):
            pv = [pos_v[i, pl.ds(j * L, L)] for j in range(D // L)]
            for bb in range(CH_B):
                r = bb * BAND + i
                for j in range(D // L):
                    sl = pl.ds(j * L, L)
                    buf[r, sl] = buf[r, sl] * SCALE + pv[j]
            return carry

        lax.fori_loop(0, BAND, body_i, 0)

        for bb in range(CH_B):
            outc[s].append(pltpu.async_copy(
                buf.at[pl.ds(bb * BAND, BAND)],
                out_hbm.at[c * CH_B + bb, pl.ds(s0, BAND), :],
                sem_o[s]))
    for hs in outc:
        for h in hs:
            h.wait()


def kernel(x, table):
    pos = jnp.asarray(_POS)
    xp = x.reshape(BATCH, NW, BAND).transpose(1, 0, 2).reshape(NW, BATCH * BAND)
    return _emb_kernel(xp, pos, table)


# speedup vs baseline: 1.8747x; 1.0147x over previous
import functools

import numpy as np
import jax
import jax.numpy as jnp
from jax import lax
from jax.experimental import pallas as pl
from jax.experimental.pallas import tpu as pltpu
from jax.experimental.pallas import tpu_sc as plsc

BATCH = 32
SEQ = 2048
D = 128
SCALE = float(np.sqrt(128.0))

_info = plsc.get_sparse_core_info()
NC, NS, L = _info.num_cores, _info.num_subcores, _info.num_lanes
NW = NC * NS
BAND = SEQ // NW
CH_B = 4
N_CHUNK = BATCH // CH_B
CH_ROWS = CH_B * BAND


def _pos_encoding() -> np.ndarray:
    positions = np.arange(SEQ)[:, np.newaxis].astype(np.float64)
    depths = np.arange(D)[np.newaxis, :] / D
    angle_rates = 1.0 / (10000.0 ** depths)
    pe = positions * angle_rates
    pe[:, 1::2] = np.cos(pe[:, 1::2])
    pe[:, 0::2] = np.sin(pe[:, 0::2])
    return pe.astype(np.float32)


_POS = _pos_encoding()

_mesh = plsc.VectorSubcoreMesh(core_axis_name="c", subcore_axis_name="s")


@functools.partial(
    pl.kernel,
    mesh=_mesh,
    out_type=jax.ShapeDtypeStruct((BATCH, SEQ, D), jnp.float32),
    scratch_types=[
        pltpu.VMEM((BATCH * BAND,), jnp.int32),
        pltpu.VMEM((BAND, D), jnp.float32),
        pltpu.VMEM((CH_ROWS, D), jnp.float32),
        pltpu.VMEM((CH_ROWS, D), jnp.float32),
        pltpu.VMEM((CH_ROWS, D), jnp.float32),
        pltpu.SemaphoreType.DMA,
        pltpu.SemaphoreType.DMA,
        pltpu.SemaphoreType.DMA,
        pltpu.SemaphoreType.DMA,
        pltpu.SemaphoreType.DMA,
        pltpu.SemaphoreType.DMA,
        pltpu.SemaphoreType.DMA,
        pltpu.SemaphoreType.DMA,
    ],
)
def _emb_kernel(xf_hbm, pos_hbm, table_hbm, out_hbm,
                idx_v, pos_v, rows0, rows1, rows2,
                sem_i, sem_p, sem_g0, sem_g1, sem_g2, sem_o0, sem_o1, sem_o2):
    wid = lax.axis_index("s") * NC + lax.axis_index("c")
    s0 = wid * BAND
    rows = (rows0, rows1, rows2)
    sem_g = (sem_g0, sem_g1, sem_g2)
    sem_o = (sem_o0, sem_o1, sem_o2)
    NBUF = 3

    cps_i = [
        pltpu.async_copy(
            xf_hbm.at[pl.ds(b * SEQ + s0, BAND)],
            idx_v.at[pl.ds(b * BAND, BAND)], sem_i)
        for b in range(BATCH)
    ]
    cp_p = pltpu.async_copy(pos_hbm.at[pl.ds(s0, BAND), :], pos_v, sem_p)
    for cp in cps_i:
        cp.wait()
    gath = [None, None, None]
    outc = [[], [], []]
    gath[0] = pltpu.async_copy(
        table_hbm.at[idx_v.at[pl.ds(0, CH_ROWS)]], rows[0], sem_g[0])
    gath[1] = pltpu.async_copy(
        table_hbm.at[idx_v.at[pl.ds(CH_ROWS, CH_ROWS)]], rows[1], sem_g[1])
    cp_p.wait()

    for c in range(N_CHUNK):
        s = c % NBUF
        if c + 2 < N_CHUNK:
            s2 = (c + 2) % NBUF
            for h in outc[s2]:
                h.wait()
            outc[s2] = []
            gath[s2] = pltpu.async_copy(
                table_hbm.at[idx_v.at[pl.ds((c + 2) * CH_ROWS, CH_ROWS)]],
                rows[s2], sem_g[s2])
        gath[s].wait()
        buf = rows[s]

        def body_i(i, carry, buf=buf):
            pv = [pos_v[i, pl.ds(j * L, L)] for j in range(D // L)]
            for bb in range(CH_B):
                r = bb * BAND + i
                for j in range(D // L):
                    sl = pl.ds(j * L, L)
                    buf[r, sl] = buf[r, sl] * SCALE + pv[j]
            return carry

        lax.fori_loop(0, BAND, body_i, 0)

        for bb in range(CH_B):
            outc[s].append(pltpu.async_copy(
                buf.at[pl.ds(bb * BAND, BAND)],
                out_hbm.at[c * CH_B + bb, pl.ds(s0, BAND), :],
                sem_o[s]))
    for hs in outc:
        for h in hs:
            h.wait()


def kernel(x, table):
    pos = jnp.asarray(_POS)
    return _emb_kernel(x.reshape(BATCH * SEQ), pos, table)
